# Initial kernel scaffold; baseline (speedup 1.0000x reference)
import jax
import jax.numpy as jnp
from jax.experimental import pallas as pl


def kernel(edge_attr, W0, W1, W2):
    raise NotImplementedError("write your pallas kernel here")


# speedup vs baseline: 1.7693x; 1.7693x over previous
import functools

import jax
import jax.numpy as jnp
from jax import lax
from jax.experimental import pallas as pl
from jax.experimental.pallas import tpu as pltpu
from jax.experimental.pallas import tpu_sc as plsc

N = 800000
D = 64
NP = N // 2
T2_ROWS = 3600

NC = 2
NS = 16
NW = NC * NS
C = 128
NCHT = NP // C
ITERS = -(-NCHT // NW)
GP = C // 16

_mesh = plsc.VectorSubcoreMesh(core_axis_name="c", subcore_axis_name="s")


@functools.partial(
    pl.kernel,
    out_type=jax.ShapeDtypeStruct((NP, 2 * D), jnp.float32),
    mesh=_mesh,
    scratch_types=[
        pltpu.VMEM((6, C), jnp.int32),
        pltpu.VMEM((C,), jnp.int32),
        pltpu.VMEM((C, 2 * D), jnp.float32),
        pltpu.SemaphoreType.DMA,
    ],
)
def _bond_encode(ea_hbm, t2_hbm, out_hbm, a_v, idx_v, rows_v, sem):
    wid = lax.axis_index("s") * NC + lax.axis_index("c")

    def chunk_body(i):
        cid = wid + i * NW

        @pl.when(cid < NCHT)
        def _():
            r0 = cid * C
            for k in range(6):
                pltpu.sync_copy(ea_hbm.at[k, pl.ds(r0, C)], a_v.at[k])
            for g in range(GP):
                s = pl.ds(g * 16, 16)
                ie = a_v[0, s] * 12 + a_v[1, s] * 2 + a_v[2, s]
                io = a_v[3, s] * 12 + a_v[4, s] * 2 + a_v[5, s]
                idx_v[s] = ie * 60 + io
            pltpu.async_copy(t2_hbm.at[idx_v], rows_v, sem).wait()
            pltpu.sync_copy(rows_v, out_hbm.at[pl.ds(r0, C)])

    pl.loop(0, ITERS)(chunk_body)


def kernel(edge_attr, W0, W1, W2):
    t = (W0[:, None, None, :] + W1[None, :, None, :] + W2[None, None, :, :])
    t = t.reshape(60, D)
    t2 = jnp.concatenate(
        [jnp.broadcast_to(t[:, None, :], (60, 60, D)),
         jnp.broadcast_to(t[None, :, :], (60, 60, D))], axis=-1,
    ).reshape(T2_ROWS, 2 * D)
    ea = edge_attr.reshape(NP, 2, 3)
    ea6 = jnp.concatenate([ea[:, 0, :].T, ea[:, 1, :].T], axis=0)
    out2 = _bond_encode(ea6, t2)
    return out2.reshape(N, D)


# speedup vs baseline: 2.1000x; 1.1869x over previous
import functools

import jax
import jax.numpy as jnp
from jax import lax
from jax.experimental import pallas as pl
from jax.experimental.pallas import tpu as pltpu
from jax.experimental.pallas import tpu_sc as plsc

N = 800000
D = 64
NP = N // 2
T2_ROWS = 3600

NC = 2
NS = 16
NW = NC * NS
C = 320
NCHT = NP // C
ITERS = -(-NCHT // NW)
GP = C // 16
SPLITS = ((0, 128), (128, 128), (256, 64))

_mesh = plsc.VectorSubcoreMesh(core_axis_name="c", subcore_axis_name="s")


@functools.partial(
    pl.kernel,
    out_type=jax.ShapeDtypeStruct((NP, 2 * D), jnp.float32),
    mesh=_mesh,
    scratch_types=[
        pltpu.VMEM((6, C), jnp.int32),
        pltpu.VMEM((6, C), jnp.int32),
        pltpu.VMEM((C,), jnp.int32),
        pltpu.VMEM((C,), jnp.int32),
        pltpu.VMEM((C, 2 * D), jnp.float32),
        pltpu.VMEM((C, 2 * D), jnp.float32),
        pltpu.SemaphoreType.DMA,
        pltpu.SemaphoreType.DMA,
        pltpu.SemaphoreType.DMA,
        pltpu.SemaphoreType.DMA,
        pltpu.SemaphoreType.DMA,
        pltpu.SemaphoreType.DMA,
    ],
)
def _bond_encode(ea_hbm, t2_hbm, out_hbm,
                 attr0, attr1, idx0, idx1, rows0, rows1,
                 sa0, sa1, sg0, sg1, so0, so1):
    wid = lax.axis_index("s") * NC + lax.axis_index("c")
    attr = (attr0, attr1)
    idx = (idx0, idx1)
    rows = (rows0, rows1)
    sa = (sa0, sa1)
    sg = (sg0, sg1)
    so = (so0, so1)

    def cid_of(i):
        return wid + i * NW

    def start_attr(i, b):
        pltpu.async_copy(ea_hbm.at[cid_of(i)], attr[b], sa[b])

    def wait_attr(b):
        pltpu.make_async_copy(ea_hbm.at[0], attr[b], sa[b]).wait()

    def compute_idx(b):
        a = attr[b]
        v = idx[b]
        for g in range(GP):
            s = pl.ds(g * 16, 16)
            ie = a[0, s] * 12 + a[1, s] * 2 + a[2, s]
            io = a[3, s] * 12 + a[4, s] * 2 + a[5, s]
            v[s] = ie * 60 + io

    def start_gather(b):
        for off, cnt in SPLITS:
            pltpu.async_copy(
                t2_hbm.at[idx[b].at[pl.ds(off, cnt)]],
                rows[b].at[pl.ds(off, cnt)],
                sg[b],
            )

    def wait_gather(b):
        for off, cnt in SPLITS:
            pltpu.make_async_copy(
                t2_hbm.at[pl.ds(0, cnt)],
                rows[b].at[pl.ds(off, cnt)],
                sg[b],
            ).wait()

    def start_out(i, b):
        pltpu.async_copy(rows[b], out_hbm.at[pl.ds(cid_of(i) * C, C)], so[b])

    def wait_out(b):
        pltpu.make_async_copy(rows[b], out_hbm.at[pl.ds(0, C)], so[b]).wait()

    start_attr(0, 0)
    start_attr(1, 1)

    def super_body(sv):
        for b in range(2):
            i = sv * 2 + b

            @pl.when((i < ITERS) & (cid_of(i) < NCHT))
            def _():
                wait_attr(b)
                compute_idx(b)

                @pl.when(i >= 2)
                def _():
                    wait_out(b)

                start_gather(b)

                @pl.when((i + 2 < ITERS) & (cid_of(i + 2) < NCHT))
                def _():
                    start_attr(i + 2, b)

            @pl.when((i >= 1) & (cid_of(i - 1) < NCHT))
            def _():
                wait_gather(1 - b)
                start_out(i - 1, 1 - b)

    pl.loop(0, (ITERS + 2) // 2)(super_body)

    for j in (ITERS - 2, ITERS - 1):
        @pl.when(cid_of(j) < NCHT)
        def _():
            wait_out(j % 2)


def kernel(edge_attr, W0, W1, W2):
    t = (W0[:, None, None, :] + W1[None, :, None, :] + W2[None, None, :, :])
    t = t.reshape(60, D)
    t2 = jnp.concatenate(
        [jnp.broadcast_to(t[:, None, :], (60, 60, D)),
         jnp.broadcast_to(t[None, :, :], (60, 60, D))], axis=-1,
    ).reshape(T2_ROWS, 2 * D)
    ea = edge_attr.reshape(NCHT, C, 2, 3)
    ea_cm = ea.transpose(0, 2, 3, 1).reshape(NCHT, 6, C)
    out2 = _bond_encode(ea_cm, t2)
    return out2.reshape(N, D)
